# Initial kernel scaffold; baseline (speedup 1.0000x reference)
import functools

import jax
import jax.numpy as jnp
from jax import lax
from jax.experimental import pallas as pl
from jax.experimental.pallas import tpu as pltpu
from jax.experimental.pallas import tpu_sc as plsc

N = 10000
E = 320000
D = 128
D_HID = 256
N_CLASS = 40

NC = 2
NS = 16
L = 16
NW = NC * NS
EPW = E // NW
C = 80
NCHUNK = EPW // C
RPS = N // NS
ZR = 125

_mesh = plsc.VectorSubcoreMesh(core_axis_name="c", subcore_axis_name="s")


@functools.partial(
    pl.kernel,
    out_type=jax.ShapeDtypeStruct((NC, N, L), jnp.float32),
    mesh=_mesh,
    scratch_types=[
        pltpu.VMEM((C,), jnp.int32),
        pltpu.VMEM((C, L), jnp.float32),
        pltpu.VMEM((ZR, L), jnp.float32),
        pltpu.VMEM_SHARED((N, L), jnp.float32),
    ],
)
def _deg_kernel(ei_hbm, out_hbm, dst_v, ones_v, zero_v, acc_sh):
    cid = lax.axis_index("c")
    sid = lax.axis_index("s")
    wid = sid * NC + cid

    one = jnp.full((L,), 1.0, jnp.float32)
    zero = jnp.zeros((L,), jnp.float32)

    def fill(i, _):
        ones_v[i, :] = one
        return 0

    lax.fori_loop(0, C, fill, 0)

    def zfill(i, _):
        zero_v[i, :] = zero
        return 0

    lax.fori_loop(0, ZR, zfill, 0)

    base_row = sid * RPS
    for k in range(RPS // ZR):
        pltpu.sync_copy(zero_v, acc_sh.at[pl.ds(base_row + k * ZR, ZR)])
    plsc.subcore_barrier()

    ebase = wid * EPW

    def body(i, _):
        pltpu.sync_copy(ei_hbm.at[1, pl.ds(ebase + i * C, C)], dst_v)
        pltpu.sync_copy(ones_v, acc_sh.at[dst_v], add=True)
        return 0

    lax.fori_loop(0, NCHUNK, body, 0)
    plsc.subcore_barrier()

    pltpu.sync_copy(
        acc_sh.at[pl.ds(base_row, RPS)],
        out_hbm.at[cid, pl.ds(base_row, RPS)],
    )


@functools.partial(
    pl.kernel,
    out_type=jax.ShapeDtypeStruct((NC, N, D), jnp.float32),
    mesh=_mesh,
    scratch_types=[
        pltpu.VMEM((C,), jnp.int32),
        pltpu.VMEM((C,), jnp.int32),
        pltpu.VMEM((C, D), jnp.float32),
        pltpu.VMEM((ZR, D), jnp.float32),
        pltpu.VMEM_SHARED((N, D), jnp.float32),
        pltpu.SemaphoreType.DMA,
    ],
)
def _agg_kernel(ei_hbm, xs_hbm, out_hbm, src_v, dst_v, rows_v, zero_v, acc_sh, sem):
    cid = lax.axis_index("c")
    sid = lax.axis_index("s")
    wid = sid * NC + cid

    zero = jnp.zeros((L,), jnp.float32)

    def zfill(i, _):
        for j in range(D // L):
            zero_v[i, pl.ds(j * L, L)] = zero
        return 0

    lax.fori_loop(0, ZR, zfill, 0)

    base_row = sid * RPS
    for k in range(RPS // ZR):
        pltpu.sync_copy(zero_v, acc_sh.at[pl.ds(base_row + k * ZR, ZR)])
    plsc.subcore_barrier()

    ebase = wid * EPW

    def body(i, _):
        off = ebase + i * C
        pltpu.sync_copy(ei_hbm.at[0, pl.ds(off, C)], src_v)
        pltpu.sync_copy(ei_hbm.at[1, pl.ds(off, C)], dst_v)
        pltpu.async_copy(xs_hbm.at[src_v], rows_v, sem).wait()
        pltpu.sync_copy(rows_v, acc_sh.at[dst_v], add=True)
        return 0

    lax.fori_loop(0, NCHUNK, body, 0)
    plsc.subcore_barrier()

    pltpu.sync_copy(
        acc_sh.at[pl.ds(base_row, RPS)],
        out_hbm.at[cid, pl.ds(base_row, RPS)],
    )


def _tc1_body(degp_ref, x_ref, dinv_ref, xs_ref):
    s = degp_ref[0] + degp_ref[1]
    deg = s[:, 0:1] + 1.0
    dinv = lax.rsqrt(jnp.maximum(deg, 1.0))
    dinv_ref[...] = dinv
    xs_ref[...] = x_ref[...] * dinv


def _tc1(degp, x):
    return pl.pallas_call(
        _tc1_body,
        out_shape=(
            jax.ShapeDtypeStruct((N, 1), jnp.float32),
            jax.ShapeDtypeStruct((N, D), jnp.float32),
        ),
    )(degp, x)


def _tc2_body(agg_ref, xs_ref, dinv_ref, w1_ref, b1_ref, w2_ref, ts_ref):
    dinv = dinv_ref[...]
    p = (agg_ref[0] + agg_ref[1] + xs_ref[...]) * dinv
    h1 = jnp.dot(p, w1_ref[...], preferred_element_type=jnp.float32)
    h1 = jnp.maximum(h1 + b1_ref[...], 0.0)
    t = jnp.dot(h1, w2_ref[...], preferred_element_type=jnp.float32)
    ts_ref[...] = t * dinv


def _tc2(agg1, xs, dinv, W1, b1, W2):
    return pl.pallas_call(
        _tc2_body,
        out_shape=jax.ShapeDtypeStruct((N, D), jnp.float32),
    )(agg1, xs, dinv, W1, b1, W2)


def _tc3_body(agg_ref, ts_ref, dinv_ref, b2_ref, wfc_ref, bfc_ref, out_ref):
    h2 = (agg_ref[0] + agg_ref[1] + ts_ref[...]) * dinv_ref[...]
    h2 = jnp.maximum(h2 + b2_ref[...], 0.0)
    o = jnp.dot(h2, wfc_ref[...], preferred_element_type=jnp.float32)
    out_ref[...] = o + bfc_ref[...]


def _tc3(agg2, ts, dinv, b2, Wfc, bfc):
    return pl.pallas_call(
        _tc3_body,
        out_shape=jax.ShapeDtypeStruct((N, N_CLASS), jnp.float32),
    )(agg2, ts, dinv, b2, Wfc, bfc)


@jax.jit
def kernel(x, edge_index, W1, b1, W2, b2, Wfc, bfc):
    degp = _deg_kernel(edge_index)
    dinv, xs = _tc1(degp, x)
    agg1 = _agg_kernel(edge_index, xs)
    ts = _tc2(agg1, xs, dinv, W1, b1.reshape(1, -1), W2)
    agg2 = _agg_kernel(edge_index, ts)
    return _tc3(agg2, ts, dinv, b2.reshape(1, -1), Wfc, bfc.reshape(1, -1))


# speedup vs baseline: 13.1811x; 13.1811x over previous
import functools

import jax
import jax.numpy as jnp
from jax import lax
from jax.experimental import pallas as pl
from jax.experimental.pallas import tpu as pltpu
from jax.experimental.pallas import tpu_sc as plsc

N = 10000
E = 320000
D = 128
D_HID = 256
N_CLASS = 40

NC = 2
NS = 16
L = 16
NW = NC * NS
EPW = E // NW
C = 80
NCHUNK = EPW // C
NP = 10240
RPS = NP // NS
ZR = 128

_mesh = plsc.VectorSubcoreMesh(core_axis_name="c", subcore_axis_name="s")


@functools.partial(
    pl.kernel,
    out_type=jax.ShapeDtypeStruct((NC, NP, D), jnp.float32),
    mesh=_mesh,
    scratch_types=[
        pltpu.VMEM((1, C), jnp.int32),
        pltpu.VMEM((C, D), jnp.float32),
        pltpu.VMEM_SHARED((NP, D), jnp.float32),
    ],
)
def _deg_kernel(dst_hbm, ones_hbm, zeros_hbm, out_hbm, dst_v, ones_v, acc_sh):
    cid = lax.axis_index("c")
    sid = lax.axis_index("s")
    wid = sid * NC + cid

    pltpu.sync_copy(ones_hbm, ones_v)
    base_row = sid * RPS
    for k in range(RPS // ZR):
        pltpu.sync_copy(zeros_hbm, acc_sh.at[pl.ds(base_row + k * ZR, ZR)])
    plsc.subcore_barrier()

    ebase = wid * EPW

    def body(i, _):
        pltpu.sync_copy(dst_hbm.at[pl.ds(ebase + i * C, C)], dst_v.at[0])
        pltpu.sync_copy(ones_v, acc_sh.at[dst_v.at[0]], add=True)
        return 0

    lax.fori_loop(0, NCHUNK, body, 0)
    plsc.subcore_barrier()

    pltpu.sync_copy(
        acc_sh.at[pl.ds(base_row, RPS)],
        out_hbm.at[cid, pl.ds(base_row, RPS)],
    )


@functools.partial(
    pl.kernel,
    out_type=jax.ShapeDtypeStruct((NC, NP, D), jnp.float32),
    mesh=_mesh,
    scratch_types=[
        pltpu.VMEM((1, C), jnp.int32),
        pltpu.VMEM((1, C), jnp.int32),
        pltpu.VMEM((C, D), jnp.float32),
        pltpu.VMEM_SHARED((NP, D), jnp.float32),
        pltpu.SemaphoreType.DMA,
    ],
)
def _agg_kernel(src_hbm, dst_hbm, xs_hbm, zeros_hbm, out_hbm,
                src_v, dst_v, rows_v, acc_sh, sem):
    cid = lax.axis_index("c")
    sid = lax.axis_index("s")
    wid = sid * NC + cid

    base_row = sid * RPS
    for k in range(RPS // ZR):
        pltpu.sync_copy(zeros_hbm, acc_sh.at[pl.ds(base_row + k * ZR, ZR)])
    plsc.subcore_barrier()

    ebase = wid * EPW

    def body(i, _):
        off = ebase + i * C
        pltpu.sync_copy(src_hbm.at[pl.ds(off, C)], src_v.at[0])
        pltpu.sync_copy(dst_hbm.at[pl.ds(off, C)], dst_v.at[0])
        pltpu.async_copy(xs_hbm.at[src_v.at[0]], rows_v, sem).wait()
        pltpu.sync_copy(rows_v, acc_sh.at[dst_v.at[0]], add=True)
        return 0

    lax.fori_loop(0, NCHUNK, body, 0)
    plsc.subcore_barrier()

    pltpu.sync_copy(
        acc_sh.at[pl.ds(base_row, RPS)],
        out_hbm.at[cid, pl.ds(base_row, RPS)],
    )


def _tc1_body(degp_ref, x_ref, dinv_ref, xs_ref):
    s = degp_ref[0] + degp_ref[1]
    deg = s[:N, 0:1] + 1.0
    dinv = lax.rsqrt(jnp.maximum(deg, 1.0))
    dinv_ref[...] = dinv
    xs_ref[...] = x_ref[...] * dinv


def _tc1(degp, x):
    return pl.pallas_call(
        _tc1_body,
        out_shape=(
            jax.ShapeDtypeStruct((N, 1), jnp.float32),
            jax.ShapeDtypeStruct((N, D), jnp.float32),
        ),
    )(degp, x)


def _tc2_body(agg_ref, xs_ref, dinv_ref, w1_ref, b1_ref, w2_ref, ts_ref):
    dinv = dinv_ref[...]
    a = agg_ref[0] + agg_ref[1]
    p = (a[:N] + xs_ref[...]) * dinv
    h1 = jnp.dot(p, w1_ref[...], preferred_element_type=jnp.float32)
    h1 = jnp.maximum(h1 + b1_ref[...], 0.0)
    t = jnp.dot(h1, w2_ref[...], preferred_element_type=jnp.float32)
    ts_ref[...] = t * dinv


def _tc2(agg1, xs, dinv, W1, b1, W2):
    return pl.pallas_call(
        _tc2_body,
        out_shape=jax.ShapeDtypeStruct((N, D), jnp.float32),
    )(agg1, xs, dinv, W1, b1, W2)


def _tc3_body(agg_ref, ts_ref, dinv_ref, b2_ref, wfc_ref, bfc_ref, out_ref):
    a = agg_ref[0] + agg_ref[1]
    h2 = (a[:N] + ts_ref[...]) * dinv_ref[...]
    h2 = jnp.maximum(h2 + b2_ref[...], 0.0)
    o = jnp.dot(h2, wfc_ref[...], preferred_element_type=jnp.float32)
    out_ref[...] = o + bfc_ref[...]


def _tc3(agg2, ts, dinv, b2, Wfc, bfc):
    return pl.pallas_call(
        _tc3_body,
        out_shape=jax.ShapeDtypeStruct((N, N_CLASS), jnp.float32),
    )(agg2, ts, dinv, b2, Wfc, bfc)


@jax.jit
def kernel(x, edge_index, W1, b1, W2, b2, Wfc, bfc):
    src = edge_index[0]
    dst = edge_index[1]
    ones_cd = jnp.ones((C, D), jnp.float32)
    zeros_d = jnp.zeros((ZR, D), jnp.float32)
    degp = _deg_kernel(dst, ones_cd, zeros_d)
    dinv, xs = _tc1(degp, x)
    agg1 = _agg_kernel(src, dst, xs, zeros_d)
    ts = _tc2(agg1, xs, dinv, W1, b1.reshape(1, -1), W2)
    agg2 = _agg_kernel(src, dst, ts, zeros_d)
    return _tc3(agg2, ts, dinv, b2.reshape(1, -1), Wfc, bfc.reshape(1, -1))


# speedup vs baseline: 26.9753x; 2.0465x over previous
import functools

import jax
import jax.numpy as jnp
from jax import lax
from jax.experimental import pallas as pl
from jax.experimental.pallas import tpu as pltpu
from jax.experimental.pallas import tpu_sc as plsc

N = 10000
E = 320000
D = 128
D_HID = 256
N_CLASS = 40

NC = 2
NS = 16
L = 16
NW = NC * NS
EPW = E // NW
C = 80
NCHUNK = EPW // C
NP = 10240
RPS = NP // NS
ZR = 128

_mesh = plsc.VectorSubcoreMesh(core_axis_name="c", subcore_axis_name="s")


@functools.partial(
    pl.kernel,
    out_type=jax.ShapeDtypeStruct((NC, NP, D), jnp.float32),
    mesh=_mesh,
    scratch_types=[
        pltpu.VMEM((NCHUNK, C), jnp.int32),
        pltpu.VMEM((C, D), jnp.float32),
        pltpu.VMEM_SHARED((NP, D), jnp.float32),
        pltpu.SemaphoreType.DMA,
    ],
)
def _deg_kernel(dst3_hbm, ones_hbm, zeros_hbm, out_hbm, dst_v, ones_v, acc_sh, sem_s):
    cid = lax.axis_index("c")
    sid = lax.axis_index("s")
    wid = sid * NC + cid

    pltpu.sync_copy(ones_hbm, ones_v)
    pltpu.sync_copy(dst3_hbm.at[wid], dst_v)
    base_row = sid * RPS
    for k in range(RPS // ZR):
        pltpu.sync_copy(zeros_hbm, acc_sh.at[pl.ds(base_row + k * ZR, ZR)])
    plsc.subcore_barrier()

    def body(j, _):
        pltpu.async_copy(ones_v, acc_sh.at[dst_v.at[j]], sem_s, add=True)

        @pl.when(j >= 4)
        def _():
            pltpu.make_async_copy(ones_v, acc_sh.at[dst_v.at[j - 4]], sem_s).wait()

        return 0

    lax.fori_loop(0, NCHUNK, body, 0)
    for t in range(4):
        pltpu.make_async_copy(ones_v, acc_sh.at[dst_v.at[NCHUNK - 4 + t]], sem_s).wait()
    plsc.subcore_barrier()

    pltpu.sync_copy(
        acc_sh.at[pl.ds(base_row, RPS)],
        out_hbm.at[cid, pl.ds(base_row, RPS)],
    )


@functools.partial(
    pl.kernel,
    out_type=jax.ShapeDtypeStruct((NC, NP, D), jnp.float32),
    mesh=_mesh,
    scratch_types=[
        pltpu.VMEM((EPW,), jnp.int32),
        pltpu.VMEM((NCHUNK, C), jnp.int32),
        pltpu.VMEM((2, C, D), jnp.float32),
        pltpu.VMEM_SHARED((NP, D), jnp.float32),
        pltpu.SemaphoreType.DMA,
        pltpu.SemaphoreType.DMA,
    ],
)
def _agg_kernel(src2_hbm, dst3_hbm, xs_hbm, zeros_hbm, out_hbm,
                src_v, dst_v, rows_v, acc_sh, sem_g, sem_s):
    cid = lax.axis_index("c")
    sid = lax.axis_index("s")
    wid = sid * NC + cid

    pltpu.sync_copy(src2_hbm.at[wid], src_v)
    pltpu.sync_copy(dst3_hbm.at[wid], dst_v)
    base_row = sid * RPS
    for k in range(RPS // ZR):
        pltpu.sync_copy(zeros_hbm, acc_sh.at[pl.ds(base_row + k * ZR, ZR)])
    plsc.subcore_barrier()

    pltpu.async_copy(xs_hbm.at[src_v.at[pl.ds(0, C)]], rows_v.at[0], sem_g)

    def body(j, _):
        @pl.when(j >= 1)
        def _():
            pltpu.make_async_copy(
                rows_v.at[lax.rem(j - 1, 2)],
                acc_sh.at[dst_v.at[j - 1]], sem_s).wait()

        @pl.when(j + 1 < NCHUNK)
        def _():
            pltpu.async_copy(
                xs_hbm.at[src_v.at[pl.ds((j + 1) * C, C)]],
                rows_v.at[lax.rem(j + 1, 2)], sem_g)

        pltpu.make_async_copy(
            xs_hbm.at[src_v.at[pl.ds(j * C, C)]], rows_v.at[lax.rem(j, 2)], sem_g).wait()
        pltpu.async_copy(
            rows_v.at[lax.rem(j, 2)],
            acc_sh.at[dst_v.at[j]], sem_s, add=True)
        return 0

    lax.fori_loop(0, NCHUNK, body, 0)
    pltpu.make_async_copy(
        rows_v.at[(NCHUNK - 1) % 2], acc_sh.at[dst_v.at[NCHUNK - 1]], sem_s).wait()
    plsc.subcore_barrier()

    pltpu.sync_copy(
        acc_sh.at[pl.ds(base_row, RPS)],
        out_hbm.at[cid, pl.ds(base_row, RPS)],
    )


def _tc1_body(degp_ref, x_ref, dinv_ref, xs_ref):
    s = degp_ref[0] + degp_ref[1]
    deg = s[:N, 0:1] + 1.0
    dinv = lax.rsqrt(jnp.maximum(deg, 1.0))
    dinv_ref[...] = dinv
    xs_ref[...] = x_ref[...] * dinv


def _tc1(degp, x):
    return pl.pallas_call(
        _tc1_body,
        out_shape=(
            jax.ShapeDtypeStruct((N, 1), jnp.float32),
            jax.ShapeDtypeStruct((N, D), jnp.float32),
        ),
    )(degp, x)


def _tc2_body(agg_ref, xs_ref, dinv_ref, w1_ref, b1_ref, w2_ref, ts_ref):
    dinv = dinv_ref[...]
    a = agg_ref[0] + agg_ref[1]
    p = (a[:N] + xs_ref[...]) * dinv
    h1 = jnp.dot(p, w1_ref[...], preferred_element_type=jnp.float32)
    h1 = jnp.maximum(h1 + b1_ref[...], 0.0)
    t = jnp.dot(h1, w2_ref[...], preferred_element_type=jnp.float32)
    ts_ref[...] = t * dinv


def _tc2(agg1, xs, dinv, W1, b1, W2):
    return pl.pallas_call(
        _tc2_body,
        out_shape=jax.ShapeDtypeStruct((N, D), jnp.float32),
    )(agg1, xs, dinv, W1, b1, W2)


def _tc3_body(agg_ref, ts_ref, dinv_ref, b2_ref, wfc_ref, bfc_ref, out_ref):
    a = agg_ref[0] + agg_ref[1]
    h2 = (a[:N] + ts_ref[...]) * dinv_ref[...]
    h2 = jnp.maximum(h2 + b2_ref[...], 0.0)
    o = jnp.dot(h2, wfc_ref[...], preferred_element_type=jnp.float32)
    out_ref[...] = o + bfc_ref[...]


def _tc3(agg2, ts, dinv, b2, Wfc, bfc):
    return pl.pallas_call(
        _tc3_body,
        out_shape=jax.ShapeDtypeStruct((N, N_CLASS), jnp.float32),
    )(agg2, ts, dinv, b2, Wfc, bfc)


@jax.jit
def kernel(x, edge_index, W1, b1, W2, b2, Wfc, bfc):
    src2 = edge_index[0].reshape(NW, EPW)
    dst3 = edge_index[1].reshape(NW, NCHUNK, C)
    ones_cd = jnp.ones((C, D), jnp.float32)
    zeros_d = jnp.zeros((ZR, D), jnp.float32)
    degp = _deg_kernel(dst3, ones_cd, zeros_d)
    dinv, xs = _tc1(degp, x)
    agg1 = _agg_kernel(src2, dst3, xs, zeros_d)
    ts = _tc2(agg1, xs, dinv, W1, b1.reshape(1, -1), W2)
    agg2 = _agg_kernel(src2, dst3, ts, zeros_d)
    return _tc3(agg2, ts, dinv, b2.reshape(1, -1), Wfc, bfc.reshape(1, -1))
